# baseline (speedup 1.0000x reference)
import functools

import jax
import jax.numpy as jnp
from jax import lax
from jax.experimental import pallas as pl
from jax.experimental.pallas import tpu as pltpu
from jax.experimental.pallas import tpu_sc as plsc

NC = 2
NS = 16
L = 16
NW = NC * NS

K_EDGE = 128
NB = 2
ZR = 64
BM = 1000

S_CHUNKS = 160
SPLIT_DEG = (88, 72)

SC0_CHUNKS = 160
SC1_CHUNKS = 0
SC1_TILES = 0
N_ROWS_SCAT = NS * SC0_CHUNKS + SC1_TILES * SC1_CHUNKS


def _mesh():
    return plsc.VectorSubcoreMesh(
        core_axis_name="c", subcore_axis_name="s", num_cores=NC, num_subcores=NS
    )


@functools.lru_cache(None)
def _make_deg_kernel(n0, n1, n_pad, K):
    rpt = n_pad // NS
    QD = 8
    nmax = max(n0, n1)

    @functools.partial(
        pl.kernel,
        out_type=jax.ShapeDtypeStruct((NC, n_pad, L), jnp.float32),
        mesh=_mesh(),
        scratch_types=[
            pltpu.VMEM((nmax, K), jnp.int32),
            pltpu.VMEM((K, L), jnp.float32),
            pltpu.VMEM((ZR, L), jnp.float32),
            pltpu.VMEM_SHARED((n_pad, L), jnp.float32),
            pltpu.SemaphoreType.DMA,
        ],
        compiler_params=pltpu.CompilerParams(use_tc_tiling_on_sc=False),
    )
    def deg_kernel(dst_hbm, out_hbm, dst_t, ones_v, zero_v, acc_sh, ssc):
        cid = lax.axis_index("c")
        sid = lax.axis_index("s")
        is0 = cid == 0
        my_n = jnp.where(is0, n0, n1)
        row_base = jnp.where(is0, sid * n0, NS * n0 + sid * n1)

        pltpu.sync_copy(dst_hbm.at[pl.ds(row_base, nmax)], dst_t)

        def fill_ones(r, _):
            ones_v[r, :] = jnp.full((L,), 1.0, jnp.float32)
            return 0

        lax.fori_loop(0, K, fill_ones, 0)

        def fill_zero(r, _):
            zero_v[r, :] = jnp.zeros((L,), jnp.float32)
            return 0

        lax.fori_loop(0, ZR, fill_zero, 0)

        base = sid * rpt

        def zero_acc(i, _):
            pltpu.sync_copy(zero_v, acc_sh.at[pl.ds(base + i * ZR, ZR)])
            return 0

        lax.fori_loop(0, rpt // ZR, zero_acc, 0)
        plsc.subcore_barrier()

        for j in range(QD):
            pltpu.async_copy(ones_v, acc_sh.at[dst_t.at[j]], ssc, add=True)

        def body(ci, _):
            pltpu.make_async_copy(ones_v, acc_sh.at[dst_t.at[ci]], ssc).wait()
            pltpu.async_copy(ones_v, acc_sh.at[dst_t.at[ci + QD]], ssc, add=True)
            return 0

        lax.fori_loop(0, my_n - QD, body, 0)

        def drain(j, _):
            pltpu.make_async_copy(
                ones_v, acc_sh.at[dst_t.at[my_n - QD + j]], ssc
            ).wait()
            return 0

        lax.fori_loop(0, QD, drain, 0)
        plsc.subcore_barrier()
        pltpu.sync_copy(
            acc_sh.at[pl.ds(base, rpt)], out_hbm.at[cid, pl.ds(base, rpt)]
        )

    return deg_kernel


@functools.lru_cache(None)
def _make_scatter_kernel(width, n_pad, K):
    rpt = n_pad // NS
    PH = 2
    PC = SC0_CHUNKS // PH

    @functools.partial(
        pl.kernel,
        out_type=jax.ShapeDtypeStruct((NC, n_pad, width), jnp.float32),
        mesh=_mesh(),
        scratch_types=[
            pltpu.VMEM((PC, K), jnp.int32),
            pltpu.VMEM((PC, K), jnp.int32),
            pltpu.VMEM((2 * NB, K, width), jnp.float32),
            pltpu.VMEM((ZR, width), jnp.float32),
            pltpu.VMEM_SHARED((n_pad, width), jnp.float32),
        ]
        + [pltpu.SemaphoreType.DMA] * (2 * NB)
        + [pltpu.SemaphoreType.DMA],
        compiler_params=pltpu.CompilerParams(use_tc_tiling_on_sc=False),
    )
    def scatter_kernel(
        g_hbm, src_hbm, dst_hbm, out_hbm, src_t, dst_t, rows_v, zero_v, acc_sh, *sems
    ):
        sg = sems[: 2 * NB]
        ssc = sems[2 * NB]
        cid = lax.axis_index("c")
        sid = lax.axis_index("s")
        is0 = cid == 0
        active1 = jnp.logical_and(cid == 1, sid < SC1_TILES)
        my_n = jnp.where(is0, SC0_CHUNKS, jnp.where(active1, SC1_CHUNKS, 0))
        my_pn = my_n // PH
        row_base = jnp.where(
            is0,
            sid * SC0_CHUNKS,
            jnp.where(active1, NS * SC0_CHUNKS + sid * SC1_CHUNKS, N_ROWS_SCAT),
        )

        def gather(ci, b):
            pltpu.async_copy(g_hbm.at[src_t.at[ci]], rows_v.at[b], sg[b])

        def gather_wait(ci, b):
            pltpu.make_async_copy(g_hbm.at[src_t.at[ci]], rows_v.at[b], sg[b]).wait()

        def scat(ci, b):
            pltpu.async_copy(rows_v.at[b], acc_sh.at[dst_t.at[ci]], ssc, add=True)

        def scat_wait(ci, b):
            pltpu.make_async_copy(rows_v.at[b], acc_sh.at[dst_t.at[ci]], ssc).wait()

        def fill_zero(r, _):
            for c in range(width // L):
                zero_v[r, pl.ds(c * L, L)] = jnp.zeros((L,), jnp.float32)
            return 0

        lax.fori_loop(0, ZR, fill_zero, 0)

        base = sid * rpt

        def zero_acc(i, _):
            pltpu.sync_copy(zero_v, acc_sh.at[pl.ds(base + i * ZR, ZR)])
            return 0

        lax.fori_loop(0, rpt // ZR, zero_acc, 0)
        plsc.subcore_barrier()

        def do_phase(phase):
            prow = row_base + phase * my_pn
            pltpu.sync_copy(src_hbm.at[pl.ds(prow, PC)], src_t)
            pltpu.sync_copy(dst_hbm.at[pl.ds(prow, PC)], dst_t)
            my_groups = my_pn // NB

            @pl.when(my_pn > 0)
            def _():
                for b in range(NB):
                    gather(b, b)

            def pair(pi, _):
                for p in (0, 1):
                    gi = 2 * pi + p
                    o = p * NB
                    oo = (1 - p) * NB
                    @pl.when(gi > 0)
                    def _():
                        for b in range(NB):
                            scat_wait(NB * (gi - 1) + b, oo + b)

                    @pl.when(gi + 1 < my_groups)
                    def _():
                        for b in range(NB):
                            gather(NB * (gi + 1) + b, oo + b)

                    for b in range(NB):
                        gather_wait(NB * gi + b, o + b)
                        scat(NB * gi + b, o + b)
                return 0

            lax.fori_loop(0, my_groups // 2, pair, 0)

            @pl.when(my_pn > 0)
            def _():
                for b in range(NB):
                    scat_wait(NB * (my_groups - 1) + b, NB + b)

        for phase in range(PH):
            do_phase(phase)

        plsc.subcore_barrier()
        pltpu.sync_copy(
            acc_sh.at[pl.ds(base, rpt)], out_hbm.at[cid, pl.ds(base, rpt)]
        )

    return scatter_kernel


def _tc1_body(x_ref, w_ref, d0_ref, d1_ref, g_ref, dinv_ref):
    deg = d0_ref[...] + d1_ref[...] + 1.0
    dinv = lax.rsqrt(jnp.maximum(deg, 1.0))
    h = jnp.dot(x_ref[...], w_ref[...], preferred_element_type=jnp.float32)
    g_ref[...] = h * dinv
    dinv_ref[...] = dinv


def _tc2_body(s0_ref, s1_ref, g_ref, dinv_ref, b_ref, w_ref, out_ref):
    dinv = dinv_ref[...]
    h = dinv * (s0_ref[...] + s1_ref[...] + g_ref[...]) + b_ref[...]
    h = jnp.maximum(h, 0.0)
    out_ref[...] = (
        jnp.dot(h, w_ref[...], preferred_element_type=jnp.float32) * dinv
    )


def _tc3_body(s0_ref, s1_ref, g_ref, dinv_ref, b_ref, out_ref):
    out_ref[...] = (
        dinv_ref[...] * (s0_ref[...] + s1_ref[...] + g_ref[...]) + b_ref[...]
    )


def kernel(x, edge_index, W1, b1, W2, b2):
    N, D = x.shape
    H = W1.shape[1]
    C = W2.shape[1]
    E = edge_index.shape[1]

    n_pad = -(-N // (NS * ZR)) * (NS * ZR)
    Cp = -(-C // L) * L

    n_rows = N_ROWS_SCAT
    assert NS * sum(SPLIT_DEG) == n_rows
    pad_rows = max(SC0_CHUNKS, *SPLIT_DEG)
    Et = (n_rows + pad_rows) * K_EDGE
    assert n_rows * K_EDGE >= E

    src = edge_index[0]
    dst = edge_index[1]
    src = jnp.concatenate([src, jnp.zeros((Et - E,), src.dtype)])
    dst = jnp.concatenate([dst, jnp.full((Et - E,), n_pad - 1, dst.dtype)])
    src2d = src.reshape(n_rows + pad_rows, K_EDGE)
    dst2d = dst.reshape(n_rows + pad_rows, K_EDGE)

    degp = _make_deg_kernel(*SPLIT_DEG, n_pad, K_EDGE)(dst2d)
    d0 = degp[0, :N, 0:1]
    d1 = degp[1, :N, 0:1]

    grid = (N // BM,)
    g1, dinv = pl.pallas_call(
        _tc1_body,
        grid=grid,
        in_specs=[
            pl.BlockSpec((BM, D), lambda i: (i, 0)),
            pl.BlockSpec((D, H), lambda i: (0, 0)),
            pl.BlockSpec((BM, 1), lambda i: (i, 0)),
            pl.BlockSpec((BM, 1), lambda i: (i, 0)),
        ],
        out_specs=[
            pl.BlockSpec((BM, H), lambda i: (i, 0)),
            pl.BlockSpec((BM, 1), lambda i: (i, 0)),
        ],
        out_shape=[
            jax.ShapeDtypeStruct((N, H), jnp.float32),
            jax.ShapeDtypeStruct((N, 1), jnp.float32),
        ],
    )(x, W1, d0, d1)

    s1 = _make_scatter_kernel(H, n_pad, K_EDGE)(g1, src2d, dst2d)

    W2p = jnp.pad(W2, ((0, 0), (0, Cp - C)))
    b1r = b1.reshape(1, H)
    g2 = pl.pallas_call(
        _tc2_body,
        grid=grid,
        in_specs=[
            pl.BlockSpec((BM, H), lambda i: (i, 0)),
            pl.BlockSpec((BM, H), lambda i: (i, 0)),
            pl.BlockSpec((BM, H), lambda i: (i, 0)),
            pl.BlockSpec((BM, 1), lambda i: (i, 0)),
            pl.BlockSpec((1, H), lambda i: (0, 0)),
            pl.BlockSpec((H, Cp), lambda i: (0, 0)),
        ],
        out_specs=pl.BlockSpec((BM, Cp), lambda i: (i, 0)),
        out_shape=jax.ShapeDtypeStruct((N, Cp), jnp.float32),
    )(s1[0, :N], s1[1, :N], g1, dinv, b1r, W2p)

    s2 = _make_scatter_kernel(Cp, n_pad, K_EDGE)(g2, src2d, dst2d)

    b2r = jnp.pad(b2, (0, Cp - C)).reshape(1, Cp)
    out = pl.pallas_call(
        _tc3_body,
        grid=grid,
        in_specs=[
            pl.BlockSpec((BM, Cp), lambda i: (i, 0)),
            pl.BlockSpec((BM, Cp), lambda i: (i, 0)),
            pl.BlockSpec((BM, Cp), lambda i: (i, 0)),
            pl.BlockSpec((BM, 1), lambda i: (i, 0)),
            pl.BlockSpec((1, Cp), lambda i: (0, 0)),
        ],
        out_specs=pl.BlockSpec((BM, Cp), lambda i: (i, 0)),
        out_shape=jax.ShapeDtypeStruct((N, Cp), jnp.float32),
    )(s2[0, :N], s2[1, :N], g2, dinv, b2r)

    return out[:, :C]


# speedup vs baseline: 1.2830x; 1.2830x over previous
import functools

import jax
import jax.numpy as jnp
from jax import lax
from jax.experimental import pallas as pl
from jax.experimental.pallas import tpu as pltpu
from jax.experimental.pallas import tpu_sc as plsc

NC = 2
NS = 16
L = 16
NW = NC * NS

K_EDGE = 128
NB = 2
ZR = 64
BM = 1000

S_CHUNKS = 160
SPLIT_DEG = (88, 72)

SC0_CHUNKS = 156
SC1_CHUNKS = 4
SC1_TILES = 16
N_ROWS_SCAT = NS * SC0_CHUNKS + SC1_TILES * SC1_CHUNKS


def _mesh():
    return plsc.VectorSubcoreMesh(
        core_axis_name="c", subcore_axis_name="s", num_cores=NC, num_subcores=NS
    )


@functools.lru_cache(None)
def _make_deg_kernel(n0, n1, n_pad, K):
    rpt = n_pad // NS
    QD = 8
    nmax = max(n0, n1)

    @functools.partial(
        pl.kernel,
        out_type=jax.ShapeDtypeStruct((NC, n_pad, L), jnp.float32),
        mesh=_mesh(),
        scratch_types=[
            pltpu.VMEM((nmax, K), jnp.int32),
            pltpu.VMEM((K, L), jnp.float32),
            pltpu.VMEM((ZR, L), jnp.float32),
            pltpu.VMEM_SHARED((n_pad, L), jnp.float32),
            pltpu.SemaphoreType.DMA,
        ],
        compiler_params=pltpu.CompilerParams(use_tc_tiling_on_sc=False),
    )
    def deg_kernel(dst_hbm, out_hbm, dst_t, ones_v, zero_v, acc_sh, ssc):
        cid = lax.axis_index("c")
        sid = lax.axis_index("s")
        is0 = cid == 0
        my_n = jnp.where(is0, n0, n1)
        row_base = jnp.where(is0, sid * n0, NS * n0 + sid * n1)

        pltpu.sync_copy(dst_hbm.at[pl.ds(row_base, nmax)], dst_t)

        def fill_ones(r, _):
            ones_v[r, :] = jnp.full((L,), 1.0, jnp.float32)
            return 0

        lax.fori_loop(0, K, fill_ones, 0)

        def fill_zero(r, _):
            zero_v[r, :] = jnp.zeros((L,), jnp.float32)
            return 0

        lax.fori_loop(0, ZR, fill_zero, 0)

        base = sid * rpt

        def zero_acc(i, _):
            pltpu.sync_copy(zero_v, acc_sh.at[pl.ds(base + i * ZR, ZR)])
            return 0

        lax.fori_loop(0, rpt // ZR, zero_acc, 0)
        plsc.subcore_barrier()

        for j in range(QD):
            pltpu.async_copy(ones_v, acc_sh.at[dst_t.at[j]], ssc, add=True)

        def body(ci, _):
            pltpu.make_async_copy(ones_v, acc_sh.at[dst_t.at[ci]], ssc).wait()
            pltpu.async_copy(ones_v, acc_sh.at[dst_t.at[ci + QD]], ssc, add=True)
            return 0

        lax.fori_loop(0, my_n - QD, body, 0)

        def drain(j, _):
            pltpu.make_async_copy(
                ones_v, acc_sh.at[dst_t.at[my_n - QD + j]], ssc
            ).wait()
            return 0

        lax.fori_loop(0, QD, drain, 0)
        plsc.subcore_barrier()
        pltpu.sync_copy(
            acc_sh.at[pl.ds(base, rpt)], out_hbm.at[cid, pl.ds(base, rpt)]
        )

    return deg_kernel


@functools.lru_cache(None)
def _make_scatter_kernel(width, n_pad, K):
    rpt = n_pad // NS
    PC = -(-SC0_CHUNKS // 2 // (2 * NB)) * (2 * NB)
    PC1 = SC0_CHUNKS - PC

    @functools.partial(
        pl.kernel,
        out_type=jax.ShapeDtypeStruct((NC, n_pad, width), jnp.float32),
        mesh=_mesh(),
        scratch_types=[
            pltpu.VMEM((PC, K), jnp.int32),
            pltpu.VMEM((PC, K), jnp.int32),
            pltpu.VMEM((2 * NB, K, width), jnp.float32),
            pltpu.VMEM((ZR, width), jnp.float32),
            pltpu.VMEM_SHARED((n_pad, width), jnp.float32),
        ]
        + [pltpu.SemaphoreType.DMA] * (2 * NB)
        + [pltpu.SemaphoreType.DMA],
        compiler_params=pltpu.CompilerParams(use_tc_tiling_on_sc=False),
    )
    def scatter_kernel(
        g_hbm, src_hbm, dst_hbm, out_hbm, src_t, dst_t, rows_v, zero_v, acc_sh, *sems
    ):
        sg = sems[: 2 * NB]
        ssc = sems[2 * NB]
        cid = lax.axis_index("c")
        sid = lax.axis_index("s")
        is0 = cid == 0
        active1 = jnp.logical_and(cid == 1, sid < SC1_TILES)
        my_n = jnp.where(is0, SC0_CHUNKS, jnp.where(active1, SC1_CHUNKS, 0))
        row_base = jnp.where(
            is0,
            sid * SC0_CHUNKS,
            jnp.where(active1, NS * SC0_CHUNKS + sid * SC1_CHUNKS, N_ROWS_SCAT),
        )

        def gather(ci, b):
            pltpu.async_copy(g_hbm.at[src_t.at[ci]], rows_v.at[b], sg[b])

        def gather_wait(ci, b):
            pltpu.make_async_copy(g_hbm.at[src_t.at[ci]], rows_v.at[b], sg[b]).wait()

        def scat(ci, b):
            pltpu.async_copy(rows_v.at[b], acc_sh.at[dst_t.at[ci]], ssc, add=True)

        def scat_wait(ci, b):
            pltpu.make_async_copy(rows_v.at[b], acc_sh.at[dst_t.at[ci]], ssc).wait()

        def fill_zero(r, _):
            for c in range(width // L):
                zero_v[r, pl.ds(c * L, L)] = jnp.zeros((L,), jnp.float32)
            return 0

        lax.fori_loop(0, ZR, fill_zero, 0)

        base = sid * rpt

        def zero_acc(i, _):
            pltpu.sync_copy(zero_v, acc_sh.at[pl.ds(base + i * ZR, ZR)])
            return 0

        lax.fori_loop(0, rpt // ZR, zero_acc, 0)
        plsc.subcore_barrier()

        def do_phase(phase, pn):
            prow = row_base + phase * pn0
            pltpu.sync_copy(src_hbm.at[pl.ds(prow, PC)], src_t)
            pltpu.sync_copy(dst_hbm.at[pl.ds(prow, PC)], dst_t)
            my_groups = pn // NB

            @pl.when(pn > 0)
            def _():
                for b in range(NB):
                    gather(b, b)

            def pair(pi, _):
                for p in (0, 1):
                    gi = 2 * pi + p
                    o = p * NB
                    oo = (1 - p) * NB
                    @pl.when(gi > 0)
                    def _():
                        for b in range(NB):
                            scat_wait(NB * (gi - 1) + b, oo + b)

                    @pl.when(gi + 1 < my_groups)
                    def _():
                        for b in range(NB):
                            gather(NB * (gi + 1) + b, oo + b)

                    for b in range(NB):
                        gather_wait(NB * gi + b, o + b)
                        scat(NB * gi + b, o + b)
                return 0

            lax.fori_loop(0, my_groups // 2, pair, 0)

            @pl.when(pn > 0)
            def _():
                for b in range(NB):
                    scat_wait(NB * (my_groups - 1) + b, NB + b)

        pn0 = jnp.where(is0, PC, SC1_CHUNKS)
        do_phase(0, pn0)
        do_phase(1, jnp.where(is0, PC1, 0))

        plsc.subcore_barrier()
        pltpu.sync_copy(
            acc_sh.at[pl.ds(base, rpt)], out_hbm.at[cid, pl.ds(base, rpt)]
        )

    return scatter_kernel


def _tc1_body(x_ref, w_ref, d0_ref, d1_ref, g_ref, dinv_ref):
    deg = d0_ref[...] + d1_ref[...] + 1.0
    dinv = lax.rsqrt(jnp.maximum(deg, 1.0))
    h = jnp.dot(x_ref[...], w_ref[...], preferred_element_type=jnp.float32)
    g_ref[...] = h * dinv
    dinv_ref[...] = dinv


def _tc2_body(s0_ref, s1_ref, g_ref, dinv_ref, b_ref, w_ref, out_ref):
    dinv = dinv_ref[...]
    h = dinv * (s0_ref[...] + s1_ref[...] + g_ref[...]) + b_ref[...]
    h = jnp.maximum(h, 0.0)
    out_ref[...] = (
        jnp.dot(h, w_ref[...], preferred_element_type=jnp.float32) * dinv
    )


def _tc3_body(s0_ref, s1_ref, g_ref, dinv_ref, b_ref, out_ref):
    out_ref[...] = (
        dinv_ref[...] * (s0_ref[...] + s1_ref[...] + g_ref[...]) + b_ref[...]
    )


def kernel(x, edge_index, W1, b1, W2, b2):
    N, D = x.shape
    H = W1.shape[1]
    C = W2.shape[1]
    E = edge_index.shape[1]

    n_pad = -(-N // (NS * ZR)) * (NS * ZR)
    Cp = -(-C // L) * L

    n_rows = N_ROWS_SCAT
    assert NS * sum(SPLIT_DEG) == n_rows
    pad_rows = max(SC0_CHUNKS, *SPLIT_DEG)
    Et = (n_rows + pad_rows) * K_EDGE
    assert n_rows * K_EDGE >= E

    src = edge_index[0]
    dst = edge_index[1]
    src = jnp.concatenate([src, jnp.zeros((Et - E,), src.dtype)])
    dst = jnp.concatenate([dst, jnp.full((Et - E,), n_pad - 1, dst.dtype)])
    src2d = src.reshape(n_rows + pad_rows, K_EDGE)
    dst2d = dst.reshape(n_rows + pad_rows, K_EDGE)

    degp = _make_deg_kernel(*SPLIT_DEG, n_pad, K_EDGE)(dst2d)
    d0 = degp[0, :N, 0:1]
    d1 = degp[1, :N, 0:1]

    grid = (N // BM,)
    g1, dinv = pl.pallas_call(
        _tc1_body,
        grid=grid,
        in_specs=[
            pl.BlockSpec((BM, D), lambda i: (i, 0)),
            pl.BlockSpec((D, H), lambda i: (0, 0)),
            pl.BlockSpec((BM, 1), lambda i: (i, 0)),
            pl.BlockSpec((BM, 1), lambda i: (i, 0)),
        ],
        out_specs=[
            pl.BlockSpec((BM, H), lambda i: (i, 0)),
            pl.BlockSpec((BM, 1), lambda i: (i, 0)),
        ],
        out_shape=[
            jax.ShapeDtypeStruct((N, H), jnp.float32),
            jax.ShapeDtypeStruct((N, 1), jnp.float32),
        ],
    )(x, W1, d0, d1)

    s1 = _make_scatter_kernel(H, n_pad, K_EDGE)(g1, src2d, dst2d)

    W2p = jnp.pad(W2, ((0, 0), (0, Cp - C)))
    b1r = b1.reshape(1, H)
    g2 = pl.pallas_call(
        _tc2_body,
        grid=grid,
        in_specs=[
            pl.BlockSpec((BM, H), lambda i: (i, 0)),
            pl.BlockSpec((BM, H), lambda i: (i, 0)),
            pl.BlockSpec((BM, H), lambda i: (i, 0)),
            pl.BlockSpec((BM, 1), lambda i: (i, 0)),
            pl.BlockSpec((1, H), lambda i: (0, 0)),
            pl.BlockSpec((H, Cp), lambda i: (0, 0)),
        ],
        out_specs=pl.BlockSpec((BM, Cp), lambda i: (i, 0)),
        out_shape=jax.ShapeDtypeStruct((N, Cp), jnp.float32),
    )(s1[0, :N], s1[1, :N], g1, dinv, b1r, W2p)

    s2 = _make_scatter_kernel(Cp, n_pad, K_EDGE)(g2, src2d, dst2d)

    b2r = jnp.pad(b2, (0, Cp - C)).reshape(1, Cp)
    out = pl.pallas_call(
        _tc3_body,
        grid=grid,
        in_specs=[
            pl.BlockSpec((BM, Cp), lambda i: (i, 0)),
            pl.BlockSpec((BM, Cp), lambda i: (i, 0)),
            pl.BlockSpec((BM, Cp), lambda i: (i, 0)),
            pl.BlockSpec((BM, 1), lambda i: (i, 0)),
            pl.BlockSpec((1, Cp), lambda i: (0, 0)),
        ],
        out_specs=pl.BlockSpec((BM, Cp), lambda i: (i, 0)),
        out_shape=jax.ShapeDtypeStruct((N, Cp), jnp.float32),
    )(s2[0, :N], s2[1, :N], g2, dinv, b2r)

    return out[:, :C]


# speedup vs baseline: 2.0284x; 1.5810x over previous
import functools

import jax
import jax.numpy as jnp
from jax import lax
from jax.experimental import pallas as pl
from jax.experimental.pallas import tpu as pltpu
from jax.experimental.pallas import tpu_sc as plsc

NC = 2
NS = 16
L = 16
NW = NC * NS

K_EDGE = 128
NB = 2
ZR = 64
BM = 2000

S_CHUNKS = 160
SPLIT_DEG = (96, 64)

SC0_CHUNKS = 152
SC1_CHUNKS = 8
SC1_TILES = 16
N_ROWS_SCAT = NS * SC0_CHUNKS + SC1_TILES * SC1_CHUNKS


def _mesh():
    return plsc.VectorSubcoreMesh(
        core_axis_name="c", subcore_axis_name="s", num_cores=NC, num_subcores=NS
    )


@functools.lru_cache(None)
def _make_deg_kernel(n0, n1, n_pad, K):
    rpt = n_pad // NS
    QD = 8
    nmax = max(n0, n1)

    @functools.partial(
        pl.kernel,
        out_type=jax.ShapeDtypeStruct((NC, n_pad, L), jnp.float32),
        mesh=_mesh(),
        scratch_types=[
            pltpu.VMEM((nmax, K), jnp.int32),
            pltpu.VMEM((K, L), jnp.float32),
            pltpu.VMEM((ZR, L), jnp.float32),
            pltpu.VMEM_SHARED((n_pad, L), jnp.float32),
            pltpu.SemaphoreType.DMA,
        ],
        compiler_params=pltpu.CompilerParams(use_tc_tiling_on_sc=False),
    )
    def deg_kernel(dst_hbm, out_hbm, dst_t, ones_v, zero_v, acc_sh, ssc):
        cid = lax.axis_index("c")
        sid = lax.axis_index("s")
        is0 = cid == 0
        my_n = jnp.where(is0, n0, n1)
        row_base = jnp.where(is0, sid * n0, NS * n0 + sid * n1)

        pltpu.sync_copy(dst_hbm.at[pl.ds(row_base, nmax)], dst_t)

        def fill_ones(r, _):
            ones_v[r, :] = jnp.full((L,), 1.0, jnp.float32)
            return 0

        lax.fori_loop(0, K, fill_ones, 0)

        def fill_zero(r, _):
            zero_v[r, :] = jnp.zeros((L,), jnp.float32)
            return 0

        lax.fori_loop(0, ZR, fill_zero, 0)

        base = sid * rpt

        def zero_acc(i, _):
            pltpu.sync_copy(zero_v, acc_sh.at[pl.ds(base + i * ZR, ZR)])
            return 0

        lax.fori_loop(0, rpt // ZR, zero_acc, 0)
        plsc.subcore_barrier()

        for j in range(QD):
            pltpu.async_copy(ones_v, acc_sh.at[dst_t.at[j]], ssc, add=True)

        def body(ci, _):
            pltpu.make_async_copy(ones_v, acc_sh.at[dst_t.at[ci]], ssc).wait()
            pltpu.async_copy(ones_v, acc_sh.at[dst_t.at[ci + QD]], ssc, add=True)
            return 0

        lax.fori_loop(0, my_n - QD, body, 0)

        def drain(j, _):
            pltpu.make_async_copy(
                ones_v, acc_sh.at[dst_t.at[my_n - QD + j]], ssc
            ).wait()
            return 0

        lax.fori_loop(0, QD, drain, 0)
        plsc.subcore_barrier()
        pltpu.sync_copy(
            acc_sh.at[pl.ds(base, rpt)], out_hbm.at[cid, pl.ds(base, rpt)]
        )

    return deg_kernel


@functools.lru_cache(None)
def _make_scatter_kernel(width, n_pad, K):
    rpt = n_pad // NS
    PC = SC0_CHUNKS

    @functools.partial(
        pl.kernel,
        out_type=jax.ShapeDtypeStruct((NC, n_pad, width), jnp.float32),
        mesh=_mesh(),
        scratch_types=[
            pltpu.VMEM((PC, K), jnp.int32),
            pltpu.VMEM((PC, K), jnp.int32),
            pltpu.VMEM((2 * NB, K, width), jnp.float32),
            pltpu.VMEM((ZR, width), jnp.float32),
            pltpu.VMEM_SHARED((n_pad, width), jnp.float32),
        ]
        + [pltpu.SemaphoreType.DMA] * (2 * NB)
        + [pltpu.SemaphoreType.DMA],
        compiler_params=pltpu.CompilerParams(use_tc_tiling_on_sc=False),
    )
    def scatter_kernel(
        g_hbm, src_hbm, dst_hbm, out_hbm, src_t, dst_t, rows_v, zero_v, acc_sh, *sems
    ):
        sg = sems[: 2 * NB]
        ssc = sems[2 * NB]
        cid = lax.axis_index("c")
        sid = lax.axis_index("s")
        is0 = cid == 0
        active1 = jnp.logical_and(cid == 1, sid < SC1_TILES)
        my_n = jnp.where(is0, SC0_CHUNKS, jnp.where(active1, SC1_CHUNKS, 0))
        row_base = jnp.where(
            is0,
            sid * SC0_CHUNKS,
            jnp.where(active1, NS * SC0_CHUNKS + sid * SC1_CHUNKS, N_ROWS_SCAT),
        )

        def gather(ci, b):
            pltpu.async_copy(g_hbm.at[src_t.at[ci]], rows_v.at[b], sg[b])

        def gather_wait(ci, b):
            pltpu.make_async_copy(g_hbm.at[src_t.at[ci]], rows_v.at[b], sg[b]).wait()

        def scat(ci, b):
            pltpu.async_copy(rows_v.at[b], acc_sh.at[dst_t.at[ci]], ssc, add=True)

        def scat_wait(ci, b):
            pltpu.make_async_copy(rows_v.at[b], acc_sh.at[dst_t.at[ci]], ssc).wait()

        def fill_zero(r, _):
            for c in range(width // L):
                zero_v[r, pl.ds(c * L, L)] = jnp.zeros((L,), jnp.float32)
            return 0

        lax.fori_loop(0, ZR, fill_zero, 0)

        base = sid * rpt

        def zero_acc(i, _):
            pltpu.sync_copy(zero_v, acc_sh.at[pl.ds(base + i * ZR, ZR)])
            return 0

        lax.fori_loop(0, rpt // ZR, zero_acc, 0)
        plsc.subcore_barrier()

        def do_phase(phase, pn):
            prow = row_base + phase * pn0
            pltpu.sync_copy(src_hbm.at[pl.ds(prow, PC)], src_t)
            pltpu.sync_copy(dst_hbm.at[pl.ds(prow, PC)], dst_t)
            my_groups = pn // NB

            @pl.when(pn > 0)
            def _():
                for b in range(NB):
                    gather(b, b)

            def pair(pi, _):
                for p in (0, 1):
                    gi = 2 * pi + p
                    o = p * NB
                    oo = (1 - p) * NB
                    @pl.when(gi > 0)
                    def _():
                        for b in range(NB):
                            scat_wait(NB * (gi - 1) + b, oo + b)

                    @pl.when(gi + 1 < my_groups)
                    def _():
                        for b in range(NB):
                            gather(NB * (gi + 1) + b, oo + b)

                    for b in range(NB):
                        gather_wait(NB * gi + b, o + b)
                        scat(NB * gi + b, o + b)
                return 0

            lax.fori_loop(0, my_groups // 2, pair, 0)

            @pl.when(pn > 0)
            def _():
                for b in range(NB):
                    scat_wait(NB * (my_groups - 1) + b, NB + b)

        pn0 = jnp.where(is0, PC, SC1_CHUNKS)
        do_phase(0, pn0)

        plsc.subcore_barrier()
        pltpu.sync_copy(
            acc_sh.at[pl.ds(base, rpt)], out_hbm.at[cid, pl.ds(base, rpt)]
        )

    return scatter_kernel


def _tc1_body(x_ref, w_ref, d0_ref, d1_ref, g_ref, dinv_ref):
    deg = d0_ref[...] + d1_ref[...] + 1.0
    dinv = lax.rsqrt(jnp.maximum(deg, 1.0))
    h = jnp.dot(x_ref[...], w_ref[...], preferred_element_type=jnp.float32)
    g_ref[...] = h * dinv
    dinv_ref[...] = dinv


def _tc2_body(s0_ref, s1_ref, g_ref, dinv_ref, b_ref, w_ref, out_ref):
    dinv = dinv_ref[...]
    h = dinv * (s0_ref[...] + s1_ref[...] + g_ref[...]) + b_ref[...]
    h = jnp.maximum(h, 0.0)
    out_ref[...] = (
        jnp.dot(h, w_ref[...], preferred_element_type=jnp.float32) * dinv
    )


def _tc3_body(s0_ref, s1_ref, g_ref, dinv_ref, b_ref, out_ref):
    out_ref[...] = (
        dinv_ref[...] * (s0_ref[...] + s1_ref[...] + g_ref[...]) + b_ref[...]
    )


def kernel(x, edge_index, W1, b1, W2, b2):
    N, D = x.shape
    H = W1.shape[1]
    C = W2.shape[1]
    E = edge_index.shape[1]

    n_pad = -(-N // (NS * ZR)) * (NS * ZR)
    Cp = -(-C // L) * L

    n_rows = N_ROWS_SCAT
    assert NS * sum(SPLIT_DEG) == n_rows
    pad_rows = max(SC0_CHUNKS, *SPLIT_DEG)
    Et = (n_rows + pad_rows) * K_EDGE
    assert n_rows * K_EDGE >= E

    src = edge_index[0]
    dst = edge_index[1]
    ar = jnp.arange(Et - E, dtype=src.dtype)
    src = jnp.concatenate([src, ar % N])
    dst = jnp.concatenate([dst, N + (ar % (n_pad - N))])
    src2d = src.reshape(n_rows + pad_rows, K_EDGE)
    dst2d = dst.reshape(n_rows + pad_rows, K_EDGE)

    degp = _make_deg_kernel(*SPLIT_DEG, n_pad, K_EDGE)(dst2d)
    d0 = degp[0, :N, 0:1]
    d1 = degp[1, :N, 0:1]

    grid = (N // BM,)
    g1, dinv = pl.pallas_call(
        _tc1_body,
        grid=grid,
        in_specs=[
            pl.BlockSpec((BM, D), lambda i: (i, 0)),
            pl.BlockSpec((D, H), lambda i: (0, 0)),
            pl.BlockSpec((BM, 1), lambda i: (i, 0)),
            pl.BlockSpec((BM, 1), lambda i: (i, 0)),
        ],
        out_specs=[
            pl.BlockSpec((BM, H), lambda i: (i, 0)),
            pl.BlockSpec((BM, 1), lambda i: (i, 0)),
        ],
        out_shape=[
            jax.ShapeDtypeStruct((N, H), jnp.float32),
            jax.ShapeDtypeStruct((N, 1), jnp.float32),
        ],
    )(x, W1, d0, d1)

    s1 = _make_scatter_kernel(H, n_pad, K_EDGE)(g1, src2d, dst2d)

    W2p = jnp.pad(W2, ((0, 0), (0, Cp - C)))
    b1r = b1.reshape(1, H)
    g2 = pl.pallas_call(
        _tc2_body,
        grid=grid,
        in_specs=[
            pl.BlockSpec((BM, H), lambda i: (i, 0)),
            pl.BlockSpec((BM, H), lambda i: (i, 0)),
            pl.BlockSpec((BM, H), lambda i: (i, 0)),
            pl.BlockSpec((BM, 1), lambda i: (i, 0)),
            pl.BlockSpec((1, H), lambda i: (0, 0)),
            pl.BlockSpec((H, Cp), lambda i: (0, 0)),
        ],
        out_specs=pl.BlockSpec((BM, Cp), lambda i: (i, 0)),
        out_shape=jax.ShapeDtypeStruct((N, Cp), jnp.float32),
    )(s1[0, :N], s1[1, :N], g1, dinv, b1r, W2p)

    s2 = _make_scatter_kernel(Cp, n_pad, K_EDGE)(g2, src2d, dst2d)

    b2r = jnp.pad(b2, (0, Cp - C)).reshape(1, Cp)
    out = pl.pallas_call(
        _tc3_body,
        grid=grid,
        in_specs=[
            pl.BlockSpec((BM, Cp), lambda i: (i, 0)),
            pl.BlockSpec((BM, Cp), lambda i: (i, 0)),
            pl.BlockSpec((BM, Cp), lambda i: (i, 0)),
            pl.BlockSpec((BM, 1), lambda i: (i, 0)),
            pl.BlockSpec((1, Cp), lambda i: (0, 0)),
        ],
        out_specs=pl.BlockSpec((BM, Cp), lambda i: (i, 0)),
        out_shape=jax.ShapeDtypeStruct((N, Cp), jnp.float32),
    )(s2[0, :N], s2[1, :N], g2, dinv, b2r)

    return out[:, :C]
